# Initial kernel scaffold; baseline (speedup 1.0000x reference)
import jax
import jax.numpy as jnp
from jax.experimental import pallas as pl


def kernel(x, wb_coarse, wb_mid, wb_fine, Wq, bq, Aq, Bq, Wk, bk, Ak, Bk, Wv, bv, Av, Bv, Wo, bo, Ao, Bo):
    raise NotImplementedError("write your pallas kernel here")


# speedup vs baseline: 6.7277x; 6.7277x over previous
import functools
import math

import jax
import jax.numpy as jnp
from jax import lax
from jax.experimental import pallas as pl
from jax.experimental.pallas import tpu as pltpu
from jax.experimental.pallas import tpu_sc as plsc

N_HEAD = 16
N_EMBD = 2048
HEAD_DIM = 128
LORA_SCALE = 2.0
B_SZ = 2
N_SEQ = 4096
M_ROWS = B_SZ * N_SEQ
CHUNK_SIZES = (256, 64, 16)
N_WORKERS = 32


def _qkv_body(x_ref, w_ref, b_ref, o_ref):
    acc = jnp.dot(x_ref[...], w_ref[0], preferred_element_type=jnp.float32)
    o_ref[0] = acc + b_ref[0, 0:1, :]


def _qkv_proj(xf, w_all, b_all):
    mblk = 256
    nblk = 1024
    grid = (3, N_EMBD // nblk, M_ROWS // mblk)
    return pl.pallas_call(
        _qkv_body,
        grid=grid,
        in_specs=[
            pl.BlockSpec((mblk, N_EMBD), lambda l, n, m: (m, 0)),
            pl.BlockSpec((1, N_EMBD, nblk), lambda l, n, m: (l, 0, n)),
            pl.BlockSpec((1, 8, nblk), lambda l, n, m: (l, 0, n)),
        ],
        out_specs=pl.BlockSpec((1, mblk, nblk), lambda l, n, m: (l, m, n)),
        out_shape=jax.ShapeDtypeStruct((3, M_ROWS, N_EMBD), jnp.float32),
    )(xf, w_all, b_all)


def _rank_body(q_ref, k_ref, o_ref):
    q = q_ref[0][:, 0:1]
    k = k_ref[0][0:1, :]
    j = lax.broadcasted_iota(jnp.int32, (1, N_SEQ), 1)
    i_glob = pl.program_id(1) * 128 + lax.broadcasted_iota(
        jnp.int32, (128, 1), 0)
    lt = (k < q).astype(jnp.int32)
    eq = jnp.where((k == q) & (j < i_glob), 1, 0)
    pos = jnp.sum(lt + eq, axis=1, keepdims=True)
    b = pl.program_id(0) % 2
    o_ref[0] = jnp.broadcast_to(pos + b * N_SEQ, (128, 8))


def _sort_ranks(wb_all6):
    wb_q = jnp.broadcast_to(wb_all6[:, :, None], (6, N_SEQ, 8))
    wb_k = jnp.broadcast_to(wb_all6[:, None, :], (6, 8, N_SEQ))
    pos8 = pl.pallas_call(
        _rank_body,
        grid=(6, N_SEQ // 128),
        in_specs=[
            pl.BlockSpec((1, 128, 8), lambda r, i: (r, i, 0)),
            pl.BlockSpec((1, 8, N_SEQ), lambda r, i: (r, 0, 0)),
        ],
        out_specs=pl.BlockSpec((1, 128, 8), lambda r, i: (r, i, 0)),
        out_shape=jax.ShapeDtypeStruct((6, N_SEQ, 8), jnp.int32),
    )(wb_q, wb_k)
    return pos8[:, :, 0].reshape(3, M_ROWS)


_SC_MESH = dict(core_axis_name="c", subcore_axis_name="s",
                num_cores=2, num_subcores=16)


def _sc_scatter_qkv(qkv_flat, pos3):
    rows_pw = (3 * M_ROWS) // N_WORKERS
    n_chunks = rows_pw // 16

    @functools.partial(
        pl.kernel,
        out_type=[jax.ShapeDtypeStruct((3 * M_ROWS, N_EMBD), jnp.float32)
                  for _ in range(3)],
        mesh=plsc.VectorSubcoreMesh(**_SC_MESH),
        scratch_types=[
            pltpu.VMEM((n_chunks, 16), jnp.int32),
            pltpu.VMEM((16, N_EMBD), jnp.float32),
            pltpu.VMEM((16, N_EMBD), jnp.float32),
            pltpu.SemaphoreType.DMA,
            pltpu.SemaphoreType.DMA,
        ],
    )
    def k(src_hbm, pos_hbm, o0, o1, o2, idx_v, buf0, buf1, sem0, sem1):
        wid = lax.axis_index("s") * 2 + lax.axis_index("c")
        base = wid * rows_pw
        bufs = (buf0, buf1)
        sems = (sem0, sem1)
        for lvl, dst in enumerate((o0, o1, o2)):
            pltpu.sync_copy(pos_hbm.at[lvl, wid], idx_v)

            def pair_body(i, _, dst=dst):
                for b in (0, 1):
                    c = i * 2 + b

                    @pl.when(i > 0)
                    def _():
                        pltpu.make_async_copy(
                            src_hbm.at[pl.ds(0, 16)], bufs[b],
                            sems[b]).wait()

                    pltpu.sync_copy(
                        src_hbm.at[pl.ds(base + c * 16, 16)], bufs[b])
                    pltpu.async_copy(bufs[b], dst.at[idx_v[c]], sems[b])
                return 0

            lax.fori_loop(0, n_chunks // 2, pair_body, 0, unroll=False)
            for b in (0, 1):
                pltpu.make_async_copy(
                    src_hbm.at[pl.ds(0, 16)], bufs[b], sems[b]).wait()

    return k(qkv_flat, pos3)


def _attn_body(q_ref, kp_ref, kc_ref, vp_ref, vc_ref, o_ref, *, cs, rblk):
    q = q_ref[...]
    kk = jnp.concatenate([kp_ref[...], kc_ref[...]], 0)
    scores = lax.dot_general(
        q, kk, (((1,), (1,)), ((), ())),
        preferred_element_type=jnp.float32) * (1.0 / math.sqrt(HEAD_DIM))
    i = pl.program_id(1)
    p = i * rblk + lax.broadcasted_iota(jnp.int32, (rblk, 1), 0)
    cp = p // cs
    g = (i - 1) * rblk + lax.broadcasted_iota(jnp.int32, (1, 2 * rblk), 1)
    valid = (g >= (cp - 1) * cs) & (g < (cp + 1) * cs) & (g >= 0)
    scores = jnp.where(valid, scores, -1e30)
    m = jnp.max(scores, axis=1, keepdims=True)
    e = jnp.exp(scores - m)
    attn = e / jnp.sum(e, axis=1, keepdims=True)
    vv = jnp.concatenate([vp_ref[...], vc_ref[...]], 0)
    o_ref[...] = jnp.dot(attn, vv, preferred_element_type=jnp.float32)


def _attention_level(qkv_s, cs):
    rblk = max(cs, 128)
    nb = N_SEQ // rblk
    koff = M_ROWS // rblk
    voff = 2 * koff

    def q_map(b, i, h):
        return (b * nb + i, h)

    def kp_map(b, i, h):
        return (koff + b * nb + jnp.maximum(i - 1, 0), h)

    def kc_map(b, i, h):
        return (koff + b * nb + i, h)

    def vp_map(b, i, h):
        return (voff + b * nb + jnp.maximum(i - 1, 0), h)

    def vc_map(b, i, h):
        return (voff + b * nb + i, h)

    blk = (rblk, HEAD_DIM)
    return pl.pallas_call(
        functools.partial(_attn_body, cs=cs, rblk=rblk),
        grid=(B_SZ, nb, N_HEAD),
        in_specs=[
            pl.BlockSpec(blk, q_map),
            pl.BlockSpec(blk, kp_map),
            pl.BlockSpec(blk, kc_map),
            pl.BlockSpec(blk, vp_map),
            pl.BlockSpec(blk, vc_map),
        ],
        out_specs=pl.BlockSpec(blk, q_map),
        out_shape=jax.ShapeDtypeStruct((M_ROWS, N_EMBD), jnp.float32),
    )(qkv_s, qkv_s, qkv_s, qkv_s, qkv_s)


def _sc_unsort(s0, s1, s2, posg):
    rows_pw = M_ROWS // N_WORKERS
    n_chunks = rows_pw // 16

    @functools.partial(
        pl.kernel,
        out_type=[jax.ShapeDtypeStruct((M_ROWS, N_EMBD), jnp.float32)
                  for _ in range(3)],
        mesh=plsc.VectorSubcoreMesh(**_SC_MESH),
        scratch_types=[
            pltpu.VMEM((n_chunks, 16), jnp.int32),
            pltpu.VMEM((16, N_EMBD), jnp.float32),
            pltpu.VMEM((16, N_EMBD), jnp.float32),
            pltpu.SemaphoreType.DMA,
            pltpu.SemaphoreType.DMA,
        ],
    )
    def k(a0, a1, a2, pos_hbm, o0, o1, o2, idx_v, buf0, buf1, sem0, sem1):
        wid = lax.axis_index("s") * 2 + lax.axis_index("c")
        base = wid * rows_pw
        bufs = (buf0, buf1)
        sems = (sem0, sem1)
        for lvl, (src, dst) in enumerate(((a0, o0), (a1, o1), (a2, o2))):
            pltpu.sync_copy(pos_hbm.at[lvl, wid], idx_v)
            pltpu.async_copy(src.at[idx_v[0]], buf0, sem0)

            def pair_body(i, _, src=src, dst=dst):
                for b in (0, 1):
                    c = i * 2 + b

                    @pl.when(c + 1 < n_chunks)
                    def _():
                        pltpu.async_copy(
                            src.at[idx_v[c + 1]], bufs[1 - b], sems[1 - b])

                    pltpu.make_async_copy(
                        src.at[pl.ds(0, 16)], bufs[b], sems[b]).wait()
                    pltpu.sync_copy(
                        bufs[b], dst.at[pl.ds(base + c * 16, 16)])
                return 0

            lax.fori_loop(0, n_chunks // 2, pair_body, 0, unroll=False)

    return k(s0, s1, s2, posg)


def _oproj_body(g0_ref, g1_ref, g2_ref, w_ref, b_ref, o_ref):
    s = (g0_ref[...] + g1_ref[...] + g2_ref[...]) * (1.0 / 3.0)
    o_ref[...] = (jnp.dot(s, w_ref[...], preferred_element_type=jnp.float32)
                  + b_ref[0:1, :])


def _out_proj(g0, g1, g2, w, b8):
    mblk = 256
    nblk = 1024
    gspec = pl.BlockSpec((mblk, N_EMBD), lambda n, m: (m, 0))
    return pl.pallas_call(
        _oproj_body,
        grid=(N_EMBD // nblk, M_ROWS // mblk),
        in_specs=[
            gspec, gspec, gspec,
            pl.BlockSpec((N_EMBD, nblk), lambda n, m: (0, n)),
            pl.BlockSpec((8, nblk), lambda n, m: (0, n)),
        ],
        out_specs=pl.BlockSpec((mblk, nblk), lambda n, m: (m, n)),
        out_shape=jax.ShapeDtypeStruct((M_ROWS, N_EMBD), jnp.float32),
    )(g0, g1, g2, w, b8)


def _eff_w(W, A, Bm):
    return W.T + LORA_SCALE * (A.T @ Bm.T)


def kernel(x, wb_coarse, wb_mid, wb_fine,
           Wq, bq, Aq, Bq, Wk, bk, Ak, Bk, Wv, bv, Av, Bv, Wo, bo, Ao, Bo):
    xf = x.reshape(M_ROWS, N_EMBD)

    w_all = jnp.stack([_eff_w(Wq, Aq, Bq), _eff_w(Wk, Ak, Bk),
                       _eff_w(Wv, Av, Bv)])
    b_all = jnp.broadcast_to(
        jnp.stack([bq, bk, bv])[:, None, :], (3, 8, N_EMBD))

    qkv = _qkv_proj(xf, w_all, b_all)

    wb_all6 = jnp.stack([wb_coarse, wb_mid, wb_fine]).reshape(6, N_SEQ)
    pos_g = _sort_ranks(wb_all6.astype(jnp.int32))

    pos3 = (pos_g[:, None, :] +
            (jnp.arange(3, dtype=jnp.int32) * M_ROWS)[None, :, None])
    pos3 = pos3.reshape(3, N_WORKERS, 48, 16)
    qkv_s0, qkv_s1, qkv_s2 = _sc_scatter_qkv(
        qkv.reshape(3 * M_ROWS, N_EMBD), pos3)

    s_levels = [
        _attention_level(qkv_s0, CHUNK_SIZES[0]),
        _attention_level(qkv_s1, CHUNK_SIZES[1]),
        _attention_level(qkv_s2, CHUNK_SIZES[2]),
    ]

    posg = pos_g.reshape(3, N_WORKERS, 16, 16)
    g0, g1, g2 = _sc_unsort(*s_levels, posg)

    w_o = _eff_w(Wo, Ao, Bo)
    b_o8 = jnp.broadcast_to(bo[None, :], (8, N_EMBD))
    y = _out_proj(g0, g1, g2, w_o, b_o8)
    return y.reshape(B_SZ, N_SEQ, N_EMBD)


# speedup vs baseline: 7.1050x; 1.0561x over previous
import functools
import math

import jax
import jax.numpy as jnp
from jax import lax
from jax.experimental import pallas as pl
from jax.experimental.pallas import tpu as pltpu
from jax.experimental.pallas import tpu_sc as plsc

N_HEAD = 16
N_EMBD = 2048
HEAD_DIM = 128
LORA_SCALE = 2.0
B_SZ = 2
N_SEQ = 4096
M_ROWS = B_SZ * N_SEQ
CHUNK_SIZES = (256, 64, 16)
N_WORKERS = 32


def _qkv_body(x_ref, w_ref, b_ref, o_ref):
    acc = jnp.dot(x_ref[...], w_ref[0], preferred_element_type=jnp.float32)
    o_ref[0] = acc + b_ref[0, 0:1, :]


def _qkv_proj(xf, w_all, b_all):
    mblk = 256
    nblk = 1024
    grid = (3, N_EMBD // nblk, M_ROWS // mblk)
    return pl.pallas_call(
        _qkv_body,
        grid=grid,
        in_specs=[
            pl.BlockSpec((mblk, N_EMBD), lambda l, n, m: (m, 0)),
            pl.BlockSpec((1, N_EMBD, nblk), lambda l, n, m: (l, 0, n)),
            pl.BlockSpec((1, 8, nblk), lambda l, n, m: (l, 0, n)),
        ],
        out_specs=pl.BlockSpec((1, mblk, nblk), lambda l, n, m: (l, m, n)),
        out_shape=jax.ShapeDtypeStruct((3, M_ROWS, N_EMBD), jnp.float32),
    )(xf, w_all, b_all)


def _rank_body(q_ref, k_ref, o_ref):
    q = q_ref[0][:, 0:1]
    k = k_ref[0][0:1, :]
    j = lax.broadcasted_iota(jnp.int32, (1, N_SEQ), 1)
    i_glob = pl.program_id(1) * 128 + lax.broadcasted_iota(
        jnp.int32, (128, 1), 0)
    lt = (k < q).astype(jnp.int32)
    eq = jnp.where((k == q) & (j < i_glob), 1, 0)
    pos = jnp.sum(lt + eq, axis=1, keepdims=True)
    b = pl.program_id(0) % 2
    o_ref[0] = jnp.broadcast_to(pos + b * N_SEQ, (128, 8))


def _sort_ranks(wb_all6):
    wb_q = jnp.broadcast_to(wb_all6[:, :, None], (6, N_SEQ, 8))
    wb_k = jnp.broadcast_to(wb_all6[:, None, :], (6, 8, N_SEQ))
    pos8 = pl.pallas_call(
        _rank_body,
        grid=(6, N_SEQ // 128),
        in_specs=[
            pl.BlockSpec((1, 128, 8), lambda r, i: (r, i, 0)),
            pl.BlockSpec((1, 8, N_SEQ), lambda r, i: (r, 0, 0)),
        ],
        out_specs=pl.BlockSpec((1, 128, 8), lambda r, i: (r, i, 0)),
        out_shape=jax.ShapeDtypeStruct((6, N_SEQ, 8), jnp.int32),
    )(wb_q, wb_k)
    return pos8[:, :, 0].reshape(3, M_ROWS)


_SC_MESH = dict(core_axis_name="c", subcore_axis_name="s",
                num_cores=2, num_subcores=16)


def _sc_scatter_qkv(qkv_flat, pos_lvl):
    rows_pw = (3 * M_ROWS) // N_WORKERS
    n_chunks = rows_pw // 16

    @functools.partial(
        pl.kernel,
        out_type=jax.ShapeDtypeStruct((3 * M_ROWS, N_EMBD), jnp.float32),
        mesh=plsc.VectorSubcoreMesh(**_SC_MESH),
        scratch_types=[
            pltpu.VMEM((n_chunks, 16), jnp.int32),
            pltpu.VMEM((16, N_EMBD), jnp.float32),
            pltpu.VMEM((16, N_EMBD), jnp.float32),
            pltpu.SemaphoreType.DMA,
            pltpu.SemaphoreType.DMA,
        ],
    )
    def k(src_hbm, pos_hbm, dst, idx_v, buf0, buf1, sem0, sem1):
        wid = lax.axis_index("s") * 2 + lax.axis_index("c")
        base = wid * rows_pw
        bufs = (buf0, buf1)
        sems = (sem0, sem1)
        pltpu.sync_copy(pos_hbm.at[wid], idx_v)

        def pair_body(i, _):
            for b in (0, 1):
                c = i * 2 + b

                @pl.when(i > 0)
                def _():
                    pltpu.make_async_copy(
                        src_hbm.at[pl.ds(0, 16)], bufs[b], sems[b]).wait()

                pltpu.sync_copy(
                    src_hbm.at[pl.ds(base + c * 16, 16)], bufs[b])
                pltpu.async_copy(bufs[b], dst.at[idx_v[c]], sems[b])
            return 0

        lax.fori_loop(0, n_chunks // 2, pair_body, 0, unroll=False)
        for b in (0, 1):
            pltpu.make_async_copy(
                src_hbm.at[pl.ds(0, 16)], bufs[b], sems[b]).wait()

    return k(qkv_flat, pos_lvl)


def _attn_body(q_ref, kp_ref, kc_ref, vp_ref, vc_ref, o_ref, *, cs, rblk):
    q = q_ref[...]
    kk = jnp.concatenate([kp_ref[...], kc_ref[...]], 0)
    scores = lax.dot_general(
        q, kk, (((1,), (1,)), ((), ())),
        preferred_element_type=jnp.float32) * (1.0 / math.sqrt(HEAD_DIM))
    i = pl.program_id(1)
    p = i * rblk + lax.broadcasted_iota(jnp.int32, (rblk, 1), 0)
    cp = p // cs
    g = (i - 1) * rblk + lax.broadcasted_iota(jnp.int32, (1, 2 * rblk), 1)
    valid = (g >= (cp - 1) * cs) & (g < (cp + 1) * cs) & (g >= 0)
    scores = jnp.where(valid, scores, -1e30)
    m = jnp.max(scores, axis=1, keepdims=True)
    e = jnp.exp(scores - m)
    attn = e / jnp.sum(e, axis=1, keepdims=True)
    vv = jnp.concatenate([vp_ref[...], vc_ref[...]], 0)
    o_ref[...] = jnp.dot(attn, vv, preferred_element_type=jnp.float32)


def _attention_level(qkv_s, cs):
    rblk = max(cs, 128)
    nb = N_SEQ // rblk
    koff = M_ROWS // rblk
    voff = 2 * koff

    def q_map(b, i, h):
        return (b * nb + i, h)

    def kp_map(b, i, h):
        return (koff + b * nb + jnp.maximum(i - 1, 0), h)

    def kc_map(b, i, h):
        return (koff + b * nb + i, h)

    def vp_map(b, i, h):
        return (voff + b * nb + jnp.maximum(i - 1, 0), h)

    def vc_map(b, i, h):
        return (voff + b * nb + i, h)

    blk = (rblk, HEAD_DIM)
    return pl.pallas_call(
        functools.partial(_attn_body, cs=cs, rblk=rblk),
        grid=(B_SZ, nb, N_HEAD),
        in_specs=[
            pl.BlockSpec(blk, q_map),
            pl.BlockSpec(blk, kp_map),
            pl.BlockSpec(blk, kc_map),
            pl.BlockSpec(blk, vp_map),
            pl.BlockSpec(blk, vc_map),
        ],
        out_specs=pl.BlockSpec(blk, q_map),
        out_shape=jax.ShapeDtypeStruct((M_ROWS, N_EMBD), jnp.float32),
    )(qkv_s, qkv_s, qkv_s, qkv_s, qkv_s)


def _sc_unsort(s_lvl, posg_lvl):
    rows_pw = M_ROWS // N_WORKERS
    n_chunks = rows_pw // 16

    @functools.partial(
        pl.kernel,
        out_type=jax.ShapeDtypeStruct((M_ROWS, N_EMBD), jnp.float32),
        mesh=plsc.VectorSubcoreMesh(**_SC_MESH),
        scratch_types=[
            pltpu.VMEM((n_chunks, 16), jnp.int32),
            pltpu.VMEM((16, N_EMBD), jnp.float32),
            pltpu.VMEM((16, N_EMBD), jnp.float32),
            pltpu.SemaphoreType.DMA,
            pltpu.SemaphoreType.DMA,
        ],
    )
    def k(src, pos_hbm, dst, idx_v, buf0, buf1, sem0, sem1):
        wid = lax.axis_index("s") * 2 + lax.axis_index("c")
        base = wid * rows_pw
        bufs = (buf0, buf1)
        sems = (sem0, sem1)
        pltpu.sync_copy(pos_hbm.at[wid], idx_v)
        pltpu.async_copy(src.at[idx_v[0]], buf0, sem0)

        def pair_body(i, _):
            for b in (0, 1):
                c = i * 2 + b

                @pl.when(c + 1 < n_chunks)
                def _():
                    pltpu.async_copy(
                        src.at[idx_v[c + 1]], bufs[1 - b], sems[1 - b])

                pltpu.make_async_copy(
                    src.at[pl.ds(0, 16)], bufs[b], sems[b]).wait()
                pltpu.sync_copy(
                    bufs[b], dst.at[pl.ds(base + c * 16, 16)])
            return 0

        lax.fori_loop(0, n_chunks // 2, pair_body, 0, unroll=False)

    return k(s_lvl, posg_lvl)


def _oproj_body(g0_ref, g1_ref, g2_ref, w_ref, b_ref, o_ref):
    s = (g0_ref[...] + g1_ref[...] + g2_ref[...]) * (1.0 / 3.0)
    o_ref[...] = (jnp.dot(s, w_ref[...], preferred_element_type=jnp.float32)
                  + b_ref[0:1, :])


def _out_proj(g0, g1, g2, w, b8):
    mblk = 256
    nblk = 1024
    gspec = pl.BlockSpec((mblk, N_EMBD), lambda n, m: (m, 0))
    return pl.pallas_call(
        _oproj_body,
        grid=(N_EMBD // nblk, M_ROWS // mblk),
        in_specs=[
            gspec, gspec, gspec,
            pl.BlockSpec((N_EMBD, nblk), lambda n, m: (0, n)),
            pl.BlockSpec((8, nblk), lambda n, m: (0, n)),
        ],
        out_specs=pl.BlockSpec((mblk, nblk), lambda n, m: (m, n)),
        out_shape=jax.ShapeDtypeStruct((M_ROWS, N_EMBD), jnp.float32),
    )(g0, g1, g2, w, b8)


def _eff_w(W, A, Bm):
    return W.T + LORA_SCALE * (A.T @ Bm.T)


def kernel(x, wb_coarse, wb_mid, wb_fine,
           Wq, bq, Aq, Bq, Wk, bk, Ak, Bk, Wv, bv, Av, Bv, Wo, bo, Ao, Bo):
    xf = x.reshape(M_ROWS, N_EMBD)

    w_all = jnp.stack([_eff_w(Wq, Aq, Bq), _eff_w(Wk, Ak, Bk),
                       _eff_w(Wv, Av, Bv)])
    b_all = jnp.broadcast_to(
        jnp.stack([bq, bk, bv])[:, None, :], (3, 8, N_EMBD))

    qkv = _qkv_proj(xf, w_all, b_all)

    wb_all6 = jnp.stack([wb_coarse, wb_mid, wb_fine]).reshape(6, N_SEQ)
    pos_g = _sort_ranks(wb_all6.astype(jnp.int32))

    pos3 = (pos_g[:, None, :] +
            (jnp.arange(3, dtype=jnp.int32) * M_ROWS)[None, :, None])
    pos3 = pos3.reshape(3, N_WORKERS, 48, 16)
    posg = pos_g.reshape(3, N_WORKERS, 16, 16)
    qkv_flat = qkv.reshape(3 * M_ROWS, N_EMBD)

    gathered = []
    for lvl in range(3):
        qkv_s = _sc_scatter_qkv(qkv_flat, pos3[lvl])
        s_lvl = _attention_level(qkv_s, CHUNK_SIZES[lvl])
        gathered.append(_sc_unsort(s_lvl, posg[lvl]))
    g0, g1, g2 = gathered

    w_o = _eff_w(Wo, Ao, Bo)
    b_o8 = jnp.broadcast_to(bo[None, :], (8, N_EMBD))
    y = _out_proj(g0, g1, g2, w_o, b_o8)
    return y.reshape(B_SZ, N_SEQ, N_EMBD)


# speedup vs baseline: 7.1715x; 1.0094x over previous
import functools
import math

import jax
import jax.numpy as jnp
from jax import lax
from jax.experimental import pallas as pl
from jax.experimental.pallas import tpu as pltpu
from jax.experimental.pallas import tpu_sc as plsc

N_HEAD = 16
N_EMBD = 2048
HEAD_DIM = 128
LORA_SCALE = 2.0
B_SZ = 2
N_SEQ = 4096
M_ROWS = B_SZ * N_SEQ
CHUNK_SIZES = (256, 64, 16)
N_WORKERS = 32


def _qkv_body(x_ref, w_ref, b_ref, o_ref):
    acc = jnp.dot(x_ref[...], w_ref[0], preferred_element_type=jnp.float32)
    o_ref[0] = acc + b_ref[0, 0:1, :]


def _bf(x):
    return x.astype(jnp.bfloat16)


def _qkv_proj(xf, w_all, b_all):
    mblk = 256
    nblk = 1024
    grid = (3, N_EMBD // nblk, M_ROWS // mblk)
    return pl.pallas_call(
        _qkv_body,
        grid=grid,
        in_specs=[
            pl.BlockSpec((mblk, N_EMBD), lambda l, n, m: (m, 0)),
            pl.BlockSpec((1, N_EMBD, nblk), lambda l, n, m: (l, 0, n)),
            pl.BlockSpec((1, 8, nblk), lambda l, n, m: (l, 0, n)),
        ],
        out_specs=pl.BlockSpec((1, mblk, nblk), lambda l, n, m: (l, m, n)),
        out_shape=jax.ShapeDtypeStruct((3, M_ROWS, N_EMBD), jnp.float32),
    )(xf, w_all, b_all)


def _rank_body(q_ref, k_ref, o_ref):
    q = q_ref[0][:, 0:1]
    k = k_ref[0][0:1, :]
    j = lax.broadcasted_iota(jnp.int32, (1, N_SEQ), 1)
    i_glob = pl.program_id(1) * 128 + lax.broadcasted_iota(
        jnp.int32, (128, 1), 0)
    lt = (k < q).astype(jnp.int32)
    eq = jnp.where((k == q) & (j < i_glob), 1, 0)
    pos = jnp.sum(lt + eq, axis=1, keepdims=True)
    b = pl.program_id(0) % 2
    o_ref[0] = jnp.broadcast_to(pos + b * N_SEQ, (128, 8))


def _sort_ranks(wb_all6):
    wb_q = jnp.broadcast_to(wb_all6[:, :, None], (6, N_SEQ, 8))
    wb_k = jnp.broadcast_to(wb_all6[:, None, :], (6, 8, N_SEQ))
    pos8 = pl.pallas_call(
        _rank_body,
        grid=(6, N_SEQ // 128),
        in_specs=[
            pl.BlockSpec((1, 128, 8), lambda r, i: (r, i, 0)),
            pl.BlockSpec((1, 8, N_SEQ), lambda r, i: (r, 0, 0)),
        ],
        out_specs=pl.BlockSpec((1, 128, 8), lambda r, i: (r, i, 0)),
        out_shape=jax.ShapeDtypeStruct((6, N_SEQ, 8), jnp.int32),
    )(wb_q, wb_k)
    return pos8[:, :, 0].reshape(3, M_ROWS)


_SC_MESH = dict(core_axis_name="c", subcore_axis_name="s",
                num_cores=2, num_subcores=16)


def _sc_scatter_qkv(qkv_flat, pos_lvl):
    rows_pw = (3 * M_ROWS) // N_WORKERS
    n_chunks = rows_pw // 16

    @functools.partial(
        pl.kernel,
        out_type=jax.ShapeDtypeStruct((3 * M_ROWS, N_EMBD), jnp.float32),
        mesh=plsc.VectorSubcoreMesh(**_SC_MESH),
        scratch_types=[
            pltpu.VMEM((n_chunks, 16), jnp.int32),
            pltpu.VMEM((16, N_EMBD), jnp.float32),
            pltpu.VMEM((16, N_EMBD), jnp.float32),
            pltpu.SemaphoreType.DMA,
            pltpu.SemaphoreType.DMA,
        ],
    )
    def k(src_hbm, pos_hbm, dst, idx_v, buf0, buf1, sem0, sem1):
        wid = lax.axis_index("s") * 2 + lax.axis_index("c")
        base = wid * rows_pw
        bufs = (buf0, buf1)
        sems = (sem0, sem1)
        pltpu.sync_copy(pos_hbm.at[wid], idx_v)

        def pair_body(i, _):
            for b in (0, 1):
                c = i * 2 + b

                @pl.when(i > 0)
                def _():
                    pltpu.make_async_copy(
                        src_hbm.at[pl.ds(0, 16)], bufs[b], sems[b]).wait()

                pltpu.sync_copy(
                    src_hbm.at[pl.ds(base + c * 16, 16)], bufs[b])
                pltpu.async_copy(bufs[b], dst.at[idx_v[c]], sems[b])
            return 0

        lax.fori_loop(0, n_chunks // 2, pair_body, 0, unroll=False)
        for b in (0, 1):
            pltpu.make_async_copy(
                src_hbm.at[pl.ds(0, 16)], bufs[b], sems[b]).wait()

    return k(qkv_flat, pos_lvl)


def _attn_body(q_ref, kp_ref, kc_ref, vp_ref, vc_ref, o_ref, *, cs, rblk):
    q = _bf(q_ref[...])
    kk = _bf(jnp.concatenate([kp_ref[...], kc_ref[...]], 0))
    scores = lax.dot_general(
        q, kk, (((1,), (1,)), ((), ())),
        preferred_element_type=jnp.float32) * (1.0 / math.sqrt(HEAD_DIM))
    i = pl.program_id(1)
    p = i * rblk + lax.broadcasted_iota(jnp.int32, (rblk, 1), 0)
    cp = p // cs
    g = (i - 1) * rblk + lax.broadcasted_iota(jnp.int32, (1, 2 * rblk), 1)
    valid = (g >= (cp - 1) * cs) & (g < (cp + 1) * cs) & (g >= 0)
    scores = jnp.where(valid, scores, -1e30)
    m = jnp.max(scores, axis=1, keepdims=True)
    e = jnp.exp(scores - m)
    attn = _bf(e / jnp.sum(e, axis=1, keepdims=True))
    vv = _bf(jnp.concatenate([vp_ref[...], vc_ref[...]], 0))
    o_ref[...] = jnp.dot(attn, vv, preferred_element_type=jnp.float32)


def _attention_level(qkv_s, cs):
    rblk = max(cs, 128)
    nb = N_SEQ // rblk
    koff = M_ROWS // rblk
    voff = 2 * koff

    def q_map(b, i, h):
        return (b * nb + i, h)

    def kp_map(b, i, h):
        return (koff + b * nb + jnp.maximum(i - 1, 0), h)

    def kc_map(b, i, h):
        return (koff + b * nb + i, h)

    def vp_map(b, i, h):
        return (voff + b * nb + jnp.maximum(i - 1, 0), h)

    def vc_map(b, i, h):
        return (voff + b * nb + i, h)

    blk = (rblk, HEAD_DIM)
    return pl.pallas_call(
        functools.partial(_attn_body, cs=cs, rblk=rblk),
        grid=(B_SZ, nb, N_HEAD),
        in_specs=[
            pl.BlockSpec(blk, q_map),
            pl.BlockSpec(blk, kp_map),
            pl.BlockSpec(blk, kc_map),
            pl.BlockSpec(blk, vp_map),
            pl.BlockSpec(blk, vc_map),
        ],
        out_specs=pl.BlockSpec(blk, q_map),
        out_shape=jax.ShapeDtypeStruct((M_ROWS, N_EMBD), jnp.float32),
    )(qkv_s, qkv_s, qkv_s, qkv_s, qkv_s)


def _sc_unsort(s_lvl, posg_lvl):
    rows_pw = M_ROWS // N_WORKERS
    n_chunks = rows_pw // 16

    @functools.partial(
        pl.kernel,
        out_type=jax.ShapeDtypeStruct((M_ROWS, N_EMBD), jnp.float32),
        mesh=plsc.VectorSubcoreMesh(**_SC_MESH),
        scratch_types=[
            pltpu.VMEM((n_chunks, 16), jnp.int32),
            pltpu.VMEM((16, N_EMBD), jnp.float32),
            pltpu.VMEM((16, N_EMBD), jnp.float32),
            pltpu.SemaphoreType.DMA,
            pltpu.SemaphoreType.DMA,
        ],
    )
    def k(src, pos_hbm, dst, idx_v, buf0, buf1, sem0, sem1):
        wid = lax.axis_index("s") * 2 + lax.axis_index("c")
        base = wid * rows_pw
        bufs = (buf0, buf1)
        sems = (sem0, sem1)
        pltpu.sync_copy(pos_hbm.at[wid], idx_v)
        pltpu.async_copy(src.at[idx_v[0]], buf0, sem0)

        def pair_body(i, _):
            for b in (0, 1):
                c = i * 2 + b

                @pl.when(c + 1 < n_chunks)
                def _():
                    pltpu.async_copy(
                        src.at[idx_v[c + 1]], bufs[1 - b], sems[1 - b])

                pltpu.make_async_copy(
                    src.at[pl.ds(0, 16)], bufs[b], sems[b]).wait()
                pltpu.sync_copy(
                    bufs[b], dst.at[pl.ds(base + c * 16, 16)])
            return 0

        lax.fori_loop(0, n_chunks // 2, pair_body, 0, unroll=False)

    return k(s_lvl, posg_lvl)


def _oproj_body(g0_ref, g1_ref, g2_ref, w_ref, b_ref, o_ref):
    s = _bf((g0_ref[...] + g1_ref[...] + g2_ref[...]) * (1.0 / 3.0))
    o_ref[...] = (jnp.dot(s, w_ref[...], preferred_element_type=jnp.float32)
                  + b_ref[0:1, :])


def _out_proj(g0, g1, g2, w, b8):
    mblk = 256
    nblk = 1024
    gspec = pl.BlockSpec((mblk, N_EMBD), lambda n, m: (m, 0))
    return pl.pallas_call(
        _oproj_body,
        grid=(N_EMBD // nblk, M_ROWS // mblk),
        in_specs=[
            gspec, gspec, gspec,
            pl.BlockSpec((N_EMBD, nblk), lambda n, m: (0, n)),
            pl.BlockSpec((8, nblk), lambda n, m: (0, n)),
        ],
        out_specs=pl.BlockSpec((mblk, nblk), lambda n, m: (m, n)),
        out_shape=jax.ShapeDtypeStruct((M_ROWS, N_EMBD), jnp.float32),
    )(g0, g1, g2, w, b8)


def _eff_w(W, A, Bm):
    return W.T + LORA_SCALE * (A.T @ Bm.T)


def kernel(x, wb_coarse, wb_mid, wb_fine,
           Wq, bq, Aq, Bq, Wk, bk, Ak, Bk, Wv, bv, Av, Bv, Wo, bo, Ao, Bo):
    xf = x.reshape(M_ROWS, N_EMBD)

    w_all = jnp.stack([_eff_w(Wq, Aq, Bq), _eff_w(Wk, Ak, Bk),
                       _eff_w(Wv, Av, Bv)])
    b_all = jnp.broadcast_to(
        jnp.stack([bq, bk, bv])[:, None, :], (3, 8, N_EMBD))

    qkv = _qkv_proj(_bf(xf), _bf(w_all), b_all)

    wb_all6 = jnp.stack([wb_coarse, wb_mid, wb_fine]).reshape(6, N_SEQ)
    pos_g = _sort_ranks(wb_all6.astype(jnp.int32))

    pos3 = (pos_g[:, None, :] +
            (jnp.arange(3, dtype=jnp.int32) * M_ROWS)[None, :, None])
    pos3 = pos3.reshape(3, N_WORKERS, 48, 16)
    posg = pos_g.reshape(3, N_WORKERS, 16, 16)
    qkv_flat = qkv.reshape(3 * M_ROWS, N_EMBD)

    gathered = []
    for lvl in range(3):
        qkv_s = _sc_scatter_qkv(qkv_flat, pos3[lvl])
        s_lvl = _attention_level(qkv_s, CHUNK_SIZES[lvl])
        gathered.append(_sc_unsort(s_lvl, posg[lvl]))
    g0, g1, g2 = gathered

    w_o = _eff_w(Wo, Ao, Bo)
    b_o8 = jnp.broadcast_to(bo[None, :], (8, N_EMBD))
    y = _out_proj(g0, g1, g2, _bf(w_o), b_o8)
    return y.reshape(B_SZ, N_SEQ, N_EMBD)
